# baseline (speedup 1.0000x reference)
import functools

import jax
import jax.numpy as jnp
from jax import lax
from jax.experimental import pallas as pl
from jax.experimental.pallas import tpu as pltpu
from jax.experimental.pallas import tpu_sc as plsc

N = 10000
E = 320000
H = 128
HE = H // 2
NL = 16
NC = 2
NS = 16
NW = NC * NS
EPW = E // NW
K = 80
NCHUNK = EPW // K
NPAD = 10240
RPT = NPAD // NS
ZCH = K
NZC = RPT // ZCH


def _node_prep_body(atom_ref, wa_ref, ba_ref, wit_ref, ha_ref, p_ref):
    ha = jnp.maximum(
        jnp.dot(atom_ref[...], wa_ref[...],
                preferred_element_type=jnp.float32) + ba_ref[...], 0.0)
    ha_ref[...] = ha
    p_ref[...] = jnp.dot(ha, wit_ref[...], preferred_element_type=jnp.float32)


def _node_prep(atom, w_atom, b_atom, wi_top):
    return pl.pallas_call(
        _node_prep_body,
        out_shape=(
            jax.ShapeDtypeStruct((N, H), jnp.float32),
            jax.ShapeDtypeStruct((N, H), jnp.float32),
        ),
    )(atom, w_atom, b_atom, wi_top)


def _edge_prep_body(ef_ref, we_ref, be_ref, wib_ref, bi_ref, q_ref):
    he = jnp.maximum(
        jnp.dot(ef_ref[...], we_ref[...],
                preferred_element_type=jnp.float32) + be_ref[...], 0.0)
    q_ref[...] = jnp.dot(he, wib_ref[...],
                         preferred_element_type=jnp.float32) + bi_ref[...]


def _edge_prep(ef, w_edge, b_edge, wi_bot, b_init):
    be = 8000
    in_feat = ef.shape[1]
    return pl.pallas_call(
        _edge_prep_body,
        grid=(E // be,),
        in_specs=[
            pl.BlockSpec((be, in_feat), lambda i: (i, 0)),
            pl.BlockSpec((in_feat, HE), lambda i: (0, 0)),
            pl.BlockSpec((1, HE), lambda i: (0, 0)),
            pl.BlockSpec((HE, H), lambda i: (0, 0)),
            pl.BlockSpec((1, H), lambda i: (0, 0)),
        ],
        out_specs=pl.BlockSpec((be, H), lambda i: (i, 0)),
        out_shape=jax.ShapeDtypeStruct((E, H), jnp.float32),
    )(ef, w_edge, b_edge, wi_bot, b_init)


def _gmm_body(agg_ref, wh_ref, g_ref):
    s = agg_ref[0, :N] + agg_ref[1, :N]
    g_ref[...] = jnp.dot(s, wh_ref[...], preferred_element_type=jnp.float32)


def _gmm(agg, w_h):
    return pl.pallas_call(
        _gmm_body,
        out_shape=jax.ShapeDtypeStruct((N, H), jnp.float32),
    )(agg, w_h)


def _readout_body(ha_ref, agg_ref, wot_ref, wob_ref, bo_ref, out_ref):
    a = agg_ref[0, :N] + agg_ref[1, :N]
    out_ref[...] = jnp.maximum(
        jnp.dot(ha_ref[...], wot_ref[...], preferred_element_type=jnp.float32)
        + jnp.dot(a, wob_ref[...], preferred_element_type=jnp.float32)
        + bo_ref[...], 0.0)


def _readout(ha, agg, wo_top, wo_bot, b_o):
    return pl.pallas_call(
        _readout_body,
        out_shape=jax.ShapeDtypeStruct((N, H), jnp.float32),
    )(ha, agg, wo_top, wo_bot, b_o)


def _edge_pass_body(write_h, *refs):
    if write_h:
        (table_hbm, stream_hbm, sd_hbm, agg_hbm, h_hbm) = refs[:5]
        rest = refs[5:]
    else:
        (table_hbm, stream_hbm, sd_hbm, agg_hbm) = refs[:4]
        h_hbm = None
        rest = refs[4:]
    sd = list(rest[0:4])
    xb = list(rest[4:6])
    gb = list(rest[6:8])
    agg_sh = rest[8]
    semi = list(rest[9:13])
    semin = list(rest[13:15])
    semsc = list(rest[15:17])
    semh = list(rest[17:19])

    cid = lax.axis_index("c")
    sid = lax.axis_index("s")
    wid = cid * NS + sid
    ebase = wid * EPW

    def issue_idx(c, s):
        pltpu.async_copy(sd_hbm.at[wid, c], sd[s], semi[s])

    def wait_idx(s):
        pltpu.make_async_copy(sd_hbm.at[wid, 0], sd[s], semi[s]).wait()

    def issue_in(c, s, b):
        pltpu.async_copy(stream_hbm.at[pl.ds(ebase + c * K, K)], xb[b],
                         semin[b])
        pltpu.async_copy(table_hbm.at[sd[s].at[0]], gb[b], semin[b])

    def wait_in(b):
        pltpu.make_async_copy(stream_hbm.at[pl.ds(0, K)], xb[b],
                              semin[b]).wait()
        pltpu.make_async_copy(stream_hbm.at[pl.ds(0, K)], gb[b],
                              semin[b]).wait()

    def drain_out(b):
        pltpu.make_async_copy(gb[b], agg_sh.at[sd[0].at[1]], semsc[b]).wait()
        if write_h:
            pltpu.make_async_copy(gb[b], h_hbm.at[pl.ds(0, K)],
                                  semh[b]).wait()

    def process(c, s, b):
        wait_in(b)

        @pl.loop(0, K, unroll=2)
        def _ed(i):
            xs = [xb[b][i, pl.ds(cc * NL, NL)] for cc in range(H // NL)]
            gs = [gb[b][i, pl.ds(cc * NL, NL)] for cc in range(H // NL)]
            for cc in range(H // NL):
                gb[b][i, pl.ds(cc * NL, NL)] = jnp.maximum(xs[cc] + gs[cc],
                                                           0.0)

        if write_h:
            pltpu.async_copy(gb[b], h_hbm.at[pl.ds(ebase + c * K, K)],
                             semh[b])
        pltpu.async_copy(gb[b], agg_sh.at[sd[s].at[1]], semsc[b], add=True)

    def slot(c, k):
        @pl.when(c + 2 < NCHUNK)
        def _ii(): issue_idx(c + 2, (k + 2) % 4)

        @pl.when(c + 1 < NCHUNK)
        def _in():
            wait_idx((k + 1) % 4)

            @pl.when(c >= 1)
            def _dr(): drain_out((k + 1) % 2)

            issue_in(c + 1, (k + 1) % 4, (k + 1) % 2)

        process(c, k % 4, k % 2)

    @pl.loop(0, ZCH)
    def _zf(i):
        for c in range(H // NL):
            gb[0][i, pl.ds(c * NL, NL)] = jnp.zeros((NL,), jnp.float32)

    @pl.loop(0, NZC)
    def _zc(j):
        pltpu.sync_copy(gb[0], agg_sh.at[pl.ds(sid * RPT + j * ZCH, ZCH)])

    issue_idx(0, 0)
    issue_idx(1, 1)
    wait_idx(0)
    issue_in(0, 0, 0)

    plsc.subcore_barrier()

    @pl.loop(0, NCHUNK - 1, step=4)
    def _ch(c0):
        for k in range(4):
            slot(c0 + k, k)

    slot(NCHUNK - 1, (NCHUNK - 1) % 4)

    drain_out(1)
    drain_out(0)

    plsc.subcore_barrier()

    @pl.loop(0, NZC)
    def _co(j):
        r = sid * RPT + j * ZCH
        pltpu.sync_copy(agg_sh.at[pl.ds(r, ZCH)], gb[0])
        pltpu.sync_copy(gb[0], agg_hbm.at[cid, pl.ds(r, ZCH)])


def _make_edge_pass(write_h):
    mesh = plsc.VectorSubcoreMesh(core_axis_name="c", subcore_axis_name="s")
    out_type = [jax.ShapeDtypeStruct((NC, NPAD, H), jnp.float32)]
    if write_h:
        out_type.append(jax.ShapeDtypeStruct((E, H), jnp.float32))
    scratch = (
        [pltpu.VMEM((2, K), jnp.int32) for _ in range(4)]
        + [pltpu.VMEM((K, H), jnp.float32) for _ in range(4)]
        + [pltpu.VMEM_SHARED((NPAD, H), jnp.float32)]
        + [pltpu.SemaphoreType.DMA for _ in range(10)]
    )
    return pl.kernel(
        functools.partial(_edge_pass_body, write_h),
        out_type=tuple(out_type) if write_h else out_type[0],
        mesh=mesh,
        scratch_types=scratch,
    )


_edge_pass_a = _make_edge_pass(True)
_edge_pass_b = _make_edge_pass(False)


def kernel(atom_feature, edge_feature, edge_index, W_atom, b_atom, W_edge,
           b_edge, W_init, b_init, W_h, W_o, b_o):
    src = edge_index[0]
    dst = edge_index[1]
    wi_top = W_init[:H]
    wi_bot = W_init[H:]
    wo_top = W_o[:H]
    wo_bot = W_o[H:]

    sd = jnp.stack([src.reshape(NW, NCHUNK, K), dst.reshape(NW, NCHUNK, K)],
                   axis=2)
    ha, p = _node_prep(atom_feature, W_atom, b_atom.reshape(1, H), wi_top)
    qb = _edge_prep(edge_feature, W_edge, b_edge.reshape(1, HE), wi_bot,
                    b_init.reshape(1, H))
    agg, h0 = _edge_pass_a(p, qb, sd)
    for _ in range(3):
        g = _gmm(agg, W_h)
        agg = _edge_pass_b(g, h0, sd)
    return _readout(ha, agg, wo_top, wo_bot, b_o.reshape(1, H))


# speedup vs baseline: 1.0184x; 1.0184x over previous
import functools

import jax
import jax.numpy as jnp
from jax import lax
from jax.experimental import pallas as pl
from jax.experimental.pallas import tpu as pltpu
from jax.experimental.pallas import tpu_sc as plsc

N = 10000
E = 320000
H = 128
HE = H // 2
NL = 16
NC = 2
NS = 16
NW = NC * NS
EPW = E // NW
K = 80
NCHUNK = EPW // K
NPAD = 10240
RPT = NPAD // NS
ZCH = K
NZC = RPT // ZCH


def _node_prep_body(atom_ref, wa_ref, ba_ref, wit_ref, ha_ref, p_ref):
    ha = jnp.maximum(
        jnp.dot(atom_ref[...], wa_ref[...],
                preferred_element_type=jnp.float32) + ba_ref[...], 0.0)
    ha_ref[...] = ha
    p_ref[...] = jnp.dot(ha, wit_ref[...], preferred_element_type=jnp.float32)


def _node_prep(atom, w_atom, b_atom, wi_top):
    return pl.pallas_call(
        _node_prep_body,
        out_shape=(
            jax.ShapeDtypeStruct((N, H), jnp.float32),
            jax.ShapeDtypeStruct((N, H), jnp.float32),
        ),
    )(atom, w_atom, b_atom, wi_top)


def _edge_prep_body(ef_ref, we_ref, be_ref, wib_ref, bi_ref, q_ref):
    he = jnp.maximum(
        jnp.dot(ef_ref[...], we_ref[...],
                preferred_element_type=jnp.float32) + be_ref[...], 0.0)
    q_ref[...] = jnp.dot(he, wib_ref[...],
                         preferred_element_type=jnp.float32) + bi_ref[...]


def _edge_prep(ef, w_edge, b_edge, wi_bot, b_init):
    be = 8000
    in_feat = ef.shape[1]
    return pl.pallas_call(
        _edge_prep_body,
        grid=(E // be,),
        in_specs=[
            pl.BlockSpec((be, in_feat), lambda i: (i, 0)),
            pl.BlockSpec((in_feat, HE), lambda i: (0, 0)),
            pl.BlockSpec((1, HE), lambda i: (0, 0)),
            pl.BlockSpec((HE, H), lambda i: (0, 0)),
            pl.BlockSpec((1, H), lambda i: (0, 0)),
        ],
        out_specs=pl.BlockSpec((be, H), lambda i: (i, 0)),
        out_shape=jax.ShapeDtypeStruct((E, H), jnp.float32),
    )(ef, w_edge, b_edge, wi_bot, b_init)


def _gmm_body(agg_ref, wh_ref, g_ref):
    s = agg_ref[0, :N] + agg_ref[1, :N]
    g_ref[...] = jnp.dot(s, wh_ref[...], preferred_element_type=jnp.float32)


def _gmm(agg, w_h):
    return pl.pallas_call(
        _gmm_body,
        out_shape=jax.ShapeDtypeStruct((N, H), jnp.float32),
    )(agg, w_h)


def _readout_body(ha_ref, agg_ref, wot_ref, wob_ref, bo_ref, out_ref):
    a = agg_ref[0, :N] + agg_ref[1, :N]
    out_ref[...] = jnp.maximum(
        jnp.dot(ha_ref[...], wot_ref[...], preferred_element_type=jnp.float32)
        + jnp.dot(a, wob_ref[...], preferred_element_type=jnp.float32)
        + bo_ref[...], 0.0)


def _readout(ha, agg, wo_top, wo_bot, b_o):
    return pl.pallas_call(
        _readout_body,
        out_shape=jax.ShapeDtypeStruct((N, H), jnp.float32),
    )(ha, agg, wo_top, wo_bot, b_o)


def _edge_pass_body(write_h, *refs):
    if write_h:
        (table_hbm, stream_hbm, src_hbm, dst_hbm, agg_hbm, h_hbm) = refs[:6]
        rest = refs[6:]
    else:
        (table_hbm, stream_hbm, src_hbm, dst_hbm, agg_hbm) = refs[:5]
        h_hbm = None
        rest = refs[5:]
    si = list(rest[0:4])
    di = list(rest[4:8])
    xb = list(rest[8:10])
    gb = list(rest[10:12])
    agg_sh = rest[12]
    semi = list(rest[13:17])
    semin = list(rest[17:19])
    semsc = list(rest[19:21])
    semh = list(rest[21:23])

    cid = lax.axis_index("c")
    sid = lax.axis_index("s")
    wid = cid * NS + sid
    ebase = wid * EPW

    def issue_idx(c, s):
        b = ebase + c * K
        pltpu.async_copy(src_hbm.at[pl.ds(b, K)], si[s], semi[s])
        pltpu.async_copy(dst_hbm.at[pl.ds(b, K)], di[s], semi[s])

    def wait_idx(s):
        pltpu.make_async_copy(src_hbm.at[pl.ds(0, K)], si[s], semi[s]).wait()
        pltpu.make_async_copy(src_hbm.at[pl.ds(0, K)], di[s], semi[s]).wait()

    def issue_in(c, s, b):
        pltpu.async_copy(stream_hbm.at[pl.ds(ebase + c * K, K)], xb[b],
                         semin[b])
        pltpu.async_copy(table_hbm.at[si[s]], gb[b], semin[b])

    def wait_in(b):
        pltpu.make_async_copy(stream_hbm.at[pl.ds(0, K)], xb[b],
                              semin[b]).wait()
        pltpu.make_async_copy(stream_hbm.at[pl.ds(0, K)], gb[b],
                              semin[b]).wait()

    def drain_out(b):
        pltpu.make_async_copy(gb[b], agg_sh.at[di[0]], semsc[b]).wait()
        if write_h:
            pltpu.make_async_copy(gb[b], h_hbm.at[pl.ds(0, K)],
                                  semh[b]).wait()

    def process(c, s, b):
        wait_in(b)

        @pl.loop(0, K, unroll=2)
        def _ed(i):
            xs = [xb[b][i, pl.ds(cc * NL, NL)] for cc in range(H // NL)]
            gs = [gb[b][i, pl.ds(cc * NL, NL)] for cc in range(H // NL)]
            for cc in range(H // NL):
                gb[b][i, pl.ds(cc * NL, NL)] = jnp.maximum(xs[cc] + gs[cc],
                                                           0.0)

        if write_h:
            pltpu.async_copy(gb[b], h_hbm.at[pl.ds(ebase + c * K, K)],
                             semh[b])
        pltpu.async_copy(gb[b], agg_sh.at[di[s]], semsc[b], add=True)

    def slot(c, k):
        @pl.when(c + 2 < NCHUNK)
        def _ii(): issue_idx(c + 2, (k + 2) % 4)

        @pl.when(c + 1 < NCHUNK)
        def _in():
            wait_idx((k + 1) % 4)

            @pl.when(c >= 1)
            def _dr(): drain_out((k + 1) % 2)

            issue_in(c + 1, (k + 1) % 4, (k + 1) % 2)

        process(c, k % 4, k % 2)

    @pl.loop(0, ZCH)
    def _zf(i):
        for c in range(H // NL):
            gb[0][i, pl.ds(c * NL, NL)] = jnp.zeros((NL,), jnp.float32)

    @pl.loop(0, NZC)
    def _zc(j):
        pltpu.sync_copy(gb[0], agg_sh.at[pl.ds(sid * RPT + j * ZCH, ZCH)])

    issue_idx(0, 0)
    issue_idx(1, 1)
    wait_idx(0)
    issue_in(0, 0, 0)

    plsc.subcore_barrier()

    @pl.loop(0, NCHUNK - 1, step=4)
    def _ch(c0):
        for k in range(4):
            slot(c0 + k, k)

    slot(NCHUNK - 1, (NCHUNK - 1) % 4)

    drain_out(1)
    drain_out(0)

    plsc.subcore_barrier()

    @pl.loop(0, NZC)
    def _co(j):
        r = sid * RPT + j * ZCH
        pltpu.sync_copy(agg_sh.at[pl.ds(r, ZCH)], gb[0])
        pltpu.sync_copy(gb[0], agg_hbm.at[cid, pl.ds(r, ZCH)])


def _make_edge_pass(write_h):
    mesh = plsc.VectorSubcoreMesh(core_axis_name="c", subcore_axis_name="s")
    out_type = [jax.ShapeDtypeStruct((NC, NPAD, H), jnp.float32)]
    if write_h:
        out_type.append(jax.ShapeDtypeStruct((E, H), jnp.float32))
    scratch = (
        [pltpu.VMEM((K,), jnp.int32) for _ in range(8)]
        + [pltpu.VMEM((K, H), jnp.float32) for _ in range(4)]
        + [pltpu.VMEM_SHARED((NPAD, H), jnp.float32)]
        + [pltpu.SemaphoreType.DMA for _ in range(10)]
    )
    return pl.kernel(
        functools.partial(_edge_pass_body, write_h),
        out_type=tuple(out_type) if write_h else out_type[0],
        mesh=mesh,
        scratch_types=scratch,
    )


_edge_pass_a = _make_edge_pass(True)
_edge_pass_b = _make_edge_pass(False)


def kernel(atom_feature, edge_feature, edge_index, W_atom, b_atom, W_edge,
           b_edge, W_init, b_init, W_h, W_o, b_o):
    src = edge_index[0]
    dst = edge_index[1]
    wi_top = W_init[:H]
    wi_bot = W_init[H:]
    wo_top = W_o[:H]
    wo_bot = W_o[H:]

    ha, p = _node_prep(atom_feature, W_atom, b_atom.reshape(1, H), wi_top)
    qb = _edge_prep(edge_feature, W_edge, b_edge.reshape(1, HE), wi_bot,
                    b_init.reshape(1, H))
    agg, h0 = _edge_pass_a(p, qb, src, dst)
    for _ in range(3):
        g = _gmm(agg, W_h)
        agg = _edge_pass_b(g, h0, src, dst)
    return _readout(ha, agg, wo_top, wo_bot, b_o.reshape(1, H))
